# Initial kernel scaffold; baseline (speedup 1.0000x reference)
import jax
import jax.numpy as jnp
from jax.experimental import pallas as pl


def kernel(sample, W, b, C):
    raise NotImplementedError("write your pallas kernel here")


# speedup vs baseline: 35.6074x; 35.6074x over previous
import jax
import jax.numpy as jnp
from jax.experimental import pallas as pl
from jax.experimental.pallas import tpu as pltpu

_SCALE = 64
_DIM = 448
_M = 1024
_HW = _SCALE * _SCALE
_TC = 512

_DN0 = (((0,), (0,)), ((), ()))


def _merge3(a, b):
    a0, a1, a2 = a
    b0, b1, b2 = b
    x = jnp.maximum(a0, b0)
    y = jnp.minimum(a1, b1)
    m0 = jnp.minimum(a0, b0)
    m1 = jnp.minimum(x, y)
    z = jnp.maximum(x, y)
    m2 = jnp.minimum(jnp.minimum(a2, b2), z)
    return m0, m1, m2


def _top3_cols(qc):
    w = qc.shape[1]
    inf8 = jnp.full((8, w), jnp.inf, jnp.float32)
    m0, m1, m2 = inf8, inf8, inf8
    for i in range(0, _M, 16):
        v1 = qc[i:i + 8]
        v2 = qc[i + 8:i + 16]
        lo = jnp.minimum(v1, v2)
        hi = jnp.maximum(v1, v2)
        xm = jnp.maximum(m0, lo)
        ym = jnp.minimum(m1, hi)
        m0 = jnp.minimum(m0, lo)
        m1 = jnp.minimum(xm, ym)
        m2 = jnp.minimum(m2, jnp.maximum(xm, ym))
    for h in (4, 2, 1):
        m0, m1, m2 = _merge3(
            (m0[:h], m1[:h], m2[:h]),
            (m0[h:2 * h], m1[h:2 * h], m2[h:2 * h]),
        )
    return m0, m1, m2


def _dsvdd_block(x_ref, w_ref, b_ref, c_ref, out_ref):
    x = x_ref[0].astype(jnp.bfloat16)
    w = w_ref[...]
    wb = (w[:_DIM] * -2.0).astype(jnp.bfloat16)
    phi2 = jax.lax.dot_general(wb, x, _DN0,
                               preferred_element_type=jnp.float32)
    w3 = (jnp.concatenate([w[_DIM:], b_ref[...]], axis=0)
          * -2.0).astype(jnp.bfloat16)
    cols = jax.lax.broadcasted_iota(jnp.int32, (1, _HW), 1)
    step = jnp.float32(2.0 / (_SCALE - 1))
    xx = (cols % _SCALE).astype(jnp.float32) * step - 1.0
    yy = (cols // _SCALE).astype(jnp.float32) * step - 1.0
    c3 = jnp.concatenate([xx, yy, jnp.ones((1, _HW), jnp.float32)],
                         axis=0).astype(jnp.bfloat16)
    phi2 = phi2 + jax.lax.dot_general(w3, c3, _DN0,
                                      preferred_element_type=jnp.float32)
    rn = jnp.sum(phi2 * phi2, axis=0, keepdims=True) * 0.25

    c = c_ref[...]
    cn = jnp.sum(c * c, axis=0, keepdims=True)
    cb = jnp.concatenate([c, cn], axis=0).astype(jnp.bfloat16)
    phib = jnp.concatenate(
        [phi2, jnp.ones((1, _HW), jnp.float32)], axis=0
    ).astype(jnp.bfloat16)

    parts = []
    for ci in range(0, _HW, _TC):
        qc = jax.lax.dot_general(cb, phib[:, ci:ci + _TC], _DN0,
                                 preferred_element_type=jnp.float32)
        parts.append(_top3_cols(qc))
    m0 = jnp.concatenate([p[0] for p in parts], axis=1)
    m1 = jnp.concatenate([p[1] for p in parts], axis=1)
    m2 = jnp.concatenate([p[2] for p in parts], axis=1)

    d0 = jnp.sqrt(jnp.maximum(rn + m0, 1e-12))
    d1 = jnp.sqrt(jnp.maximum(rn + m1, 1e-12))
    d2 = jnp.sqrt(jnp.maximum(rn + m2, 1e-12))

    s = d0 / (1.0 + jnp.exp(d0 - d1) + jnp.exp(d0 - d2))
    out_ref[...] = s.reshape(1, 1, _HW)


def kernel(sample, W, b, C):
    B = sample.shape[0]
    X = sample.reshape(B, _DIM, _HW)
    b2 = b.reshape(1, _DIM)
    out = pl.pallas_call(
        _dsvdd_block,
        grid=(B,),
        in_specs=[
            pl.BlockSpec((1, _DIM, _HW), lambda bb: (bb, 0, 0)),
            pl.BlockSpec((_DIM + 2, _DIM), lambda bb: (0, 0)),
            pl.BlockSpec((1, _DIM), lambda bb: (0, 0)),
            pl.BlockSpec((_DIM, _M), lambda bb: (0, 0)),
        ],
        out_specs=pl.BlockSpec((1, 1, _HW), lambda bb: (bb, 0, 0)),
        out_shape=jax.ShapeDtypeStruct((B, 1, _HW), jnp.float32),
        compiler_params=pltpu.CompilerParams(
            dimension_semantics=("parallel",)),
    )(X, W, b2, C)
    score = out.reshape(B, 1, _SCALE, _SCALE)
    loss = jnp.zeros((), jnp.float32)
    return (loss, score)


# speedup vs baseline: 35.6663x; 1.0017x over previous
import jax
import jax.numpy as jnp
from jax.experimental import pallas as pl
from jax.experimental.pallas import tpu as pltpu

_SCALE = 64
_DIM = 448
_M = 1024
_HW = _SCALE * _SCALE
_TC = 2048

_DN0 = (((0,), (0,)), ((), ()))


def _merge3(a, b):
    a0, a1, a2 = a
    b0, b1, b2 = b
    x = jnp.maximum(a0, b0)
    y = jnp.minimum(a1, b1)
    m0 = jnp.minimum(a0, b0)
    m1 = jnp.minimum(x, y)
    z = jnp.maximum(x, y)
    m2 = jnp.minimum(jnp.minimum(a2, b2), z)
    return m0, m1, m2


def _top3_cols(qc):
    w = qc.shape[1]
    inf8 = jnp.full((8, w), jnp.inf, jnp.float32)
    m0, m1, m2 = inf8, inf8, inf8
    for i in range(0, _M, 16):
        v1 = qc[i:i + 8]
        v2 = qc[i + 8:i + 16]
        lo = jnp.minimum(v1, v2)
        hi = jnp.maximum(v1, v2)
        xm = jnp.maximum(m0, lo)
        ym = jnp.minimum(m1, hi)
        m0 = jnp.minimum(m0, lo)
        m1 = jnp.minimum(xm, ym)
        m2 = jnp.minimum(m2, jnp.maximum(xm, ym))
    for h in (4, 2, 1):
        m0, m1, m2 = _merge3(
            (m0[:h], m1[:h], m2[:h]),
            (m0[h:2 * h], m1[h:2 * h], m2[h:2 * h]),
        )
    return m0, m1, m2


def _dsvdd_block(x_ref, w_ref, b_ref, c_ref, out_ref):
    x = x_ref[0].astype(jnp.bfloat16)
    w = w_ref[...]
    wb = (w[:_DIM] * -2.0).astype(jnp.bfloat16)
    phi2 = jax.lax.dot_general(wb, x, _DN0,
                               preferred_element_type=jnp.float32)
    w3 = (jnp.concatenate([w[_DIM:], b_ref[...]], axis=0)
          * -2.0).astype(jnp.bfloat16)
    cols = jax.lax.broadcasted_iota(jnp.int32, (1, _HW), 1)
    step = jnp.float32(2.0 / (_SCALE - 1))
    xx = (cols % _SCALE).astype(jnp.float32) * step - 1.0
    yy = (cols // _SCALE).astype(jnp.float32) * step - 1.0
    c3 = jnp.concatenate([xx, yy, jnp.ones((1, _HW), jnp.float32)],
                         axis=0).astype(jnp.bfloat16)
    phi2 = phi2 + jax.lax.dot_general(w3, c3, _DN0,
                                      preferred_element_type=jnp.float32)
    rn = jnp.sum(phi2 * phi2, axis=0, keepdims=True) * 0.25

    c = c_ref[...]
    cn = jnp.sum(c * c, axis=0, keepdims=True)
    cb = jnp.concatenate([c, cn], axis=0).astype(jnp.bfloat16)
    phib = jnp.concatenate(
        [phi2, jnp.ones((1, _HW), jnp.float32)], axis=0
    ).astype(jnp.bfloat16)

    parts = []
    for ci in range(0, _HW, _TC):
        qc = jax.lax.dot_general(cb, phib[:, ci:ci + _TC], _DN0,
                                 preferred_element_type=jnp.float32)
        parts.append(_top3_cols(qc))
    m0 = jnp.concatenate([p[0] for p in parts], axis=1)
    m1 = jnp.concatenate([p[1] for p in parts], axis=1)
    m2 = jnp.concatenate([p[2] for p in parts], axis=1)

    d0 = jnp.sqrt(jnp.maximum(rn + m0, 1e-12))
    d1 = jnp.sqrt(jnp.maximum(rn + m1, 1e-12))
    d2 = jnp.sqrt(jnp.maximum(rn + m2, 1e-12))

    s = d0 / (1.0 + jnp.exp(d0 - d1) + jnp.exp(d0 - d2))
    out_ref[...] = s.reshape(1, 1, _HW)


def kernel(sample, W, b, C):
    B = sample.shape[0]
    X = sample.reshape(B, _DIM, _HW)
    b2 = b.reshape(1, _DIM)
    out = pl.pallas_call(
        _dsvdd_block,
        grid=(B,),
        in_specs=[
            pl.BlockSpec((1, _DIM, _HW), lambda bb: (bb, 0, 0)),
            pl.BlockSpec((_DIM + 2, _DIM), lambda bb: (0, 0)),
            pl.BlockSpec((1, _DIM), lambda bb: (0, 0)),
            pl.BlockSpec((_DIM, _M), lambda bb: (0, 0)),
        ],
        out_specs=pl.BlockSpec((1, 1, _HW), lambda bb: (bb, 0, 0)),
        out_shape=jax.ShapeDtypeStruct((B, 1, _HW), jnp.float32),
        compiler_params=pltpu.CompilerParams(
            dimension_semantics=("parallel",)),
    )(X, W, b2, C)
    score = out.reshape(B, 1, _SCALE, _SCALE)
    loss = jnp.zeros((), jnp.float32)
    return (loss, score)
